# baseline (speedup 1.0000x reference)
import functools

import jax
import jax.numpy as jnp
from jax import lax
from jax.experimental import pallas as pl
from jax.experimental.pallas import tpu as pltpu
from jax.experimental.pallas import tpu_sc as plsc

N = 10000
E = 320000
H = 128
OUT = 40
G = 64
EPS = 1e-5

NC = 2
NS = 16
NW = NC * NS

CHUNK = 128
CHUNKS_PER_TILE = 80
CHUNKS_CORE0 = 32
CHUNKS_CORE1 = 128
EDGES_PER_TILE = CHUNK * CHUNKS_PER_TILE
E_PAD = CHUNK * NS * (CHUNKS_CORE0 + CHUNKS_CORE1)
NPAD = 10112
ROWS_PER_TILE = NPAD // NS
TRASH = N


def _sc_degree_body(dstm_hbm, out_hbm, idx_v, val_v, hist_sh):
    cid = lax.axis_index("c")
    sid = lax.axis_index("s")
    wid = cid * NS + sid

    @pl.loop(0, CHUNK)
    def _zero(i):
        for j in range(H // 16):
            val_v[i, pl.ds(j * 16, 16)] = jnp.zeros((16,), jnp.float32)

    r0 = sid * ROWS_PER_TILE
    for k in range(4):
        pltpu.sync_copy(val_v, hist_sh.at[pl.ds(r0 + k * 128, 128)])
    pltpu.sync_copy(val_v.at[pl.ds(0, ROWS_PER_TILE - 512)],
                    hist_sh.at[pl.ds(r0 + 512, ROWS_PER_TILE - 512)])
    plsc.subcore_barrier()

    @pl.loop(0, CHUNK)
    def _ones(i):
        val_v[i, pl.ds(0, 16)] = jnp.full((16,), 1.0, jnp.float32)

    pltpu.sync_copy(dstm_hbm.at[pl.ds(wid * CHUNKS_PER_TILE, CHUNKS_PER_TILE)],
                    idx_v)

    @pl.loop(0, CHUNKS_PER_TILE)
    def _accum(ch):
        pltpu.sync_copy(val_v, hist_sh.at[idx_v.at[ch]], add=True)

    plsc.subcore_barrier()
    pltpu.sync_copy(hist_sh.at[pl.ds(r0, ROWS_PER_TILE)],
                    out_hbm.at[cid, pl.ds(r0, ROWS_PER_TILE)])


def _sc_scatter_body(u_hbm, srcm_hbm, dstm_hbm, out_hbm,
                     src_v, dst_v, row_a, row_b, acc_sh, sem_a, sem_b):
    cid = lax.axis_index("c")
    sid = lax.axis_index("s")
    wid = cid * NS + sid

    @pl.loop(0, CHUNK)
    def _zero(i):
        for j in range(H // 16):
            row_a[i, pl.ds(j * 16, 16)] = jnp.zeros((16,), jnp.float32)

    r0 = sid * ROWS_PER_TILE
    for k in range(4):
        pltpu.sync_copy(row_a, acc_sh.at[pl.ds(r0 + k * 128, 128)])
    pltpu.sync_copy(row_a.at[pl.ds(0, ROWS_PER_TILE - 512)],
                    acc_sh.at[pl.ds(r0 + 512, ROWS_PER_TILE - 512)])
    plsc.subcore_barrier()

    def _gather_start(ch, buf, sem):
        pltpu.async_copy(u_hbm.at[src_v.at[ch]], buf, sem)

    def _gather_wait(ch, buf, sem):
        pltpu.make_async_copy(u_hbm.at[src_v.at[ch]], buf, sem).wait()

    def _run_edges(nchunks, e0):
        q = nchunks // 4
        for hb in range(0, nchunks, q):
            pltpu.sync_copy(srcm_hbm.at[pl.ds(e0 + hb, q)],
                            src_v.at[pl.ds(0, q)])
            pltpu.sync_copy(dstm_hbm.at[pl.ds(e0 + hb, q)],
                            dst_v.at[pl.ds(0, q)])
            _gather_start(0, row_a, sem_a)

            @pl.loop(0, q // 2)
            def _edges(p):
                ch = p * 2
                _gather_start(ch + 1, row_b, sem_b)
                _gather_wait(ch, row_a, sem_a)
                pltpu.sync_copy(row_a, acc_sh.at[dst_v.at[ch]], add=True)

                @pl.when(ch + 2 < q)
                def _():
                    _gather_start(ch + 2, row_a, sem_a)

                _gather_wait(ch + 1, row_b, sem_b)
                pltpu.sync_copy(row_b, acc_sh.at[dst_v.at[ch + 1]], add=True)

    @pl.when(cid == 0)
    def _core0():
        _run_edges(CHUNKS_CORE0, sid * CHUNKS_CORE0)

    @pl.when(cid == 1)
    def _core1():
        _run_edges(CHUNKS_CORE1, NS * CHUNKS_CORE0 + sid * CHUNKS_CORE1)

    plsc.subcore_barrier()
    pltpu.sync_copy(acc_sh.at[pl.ds(r0, ROWS_PER_TILE)],
                    out_hbm.at[cid, pl.ds(r0, ROWS_PER_TILE)])


def _bn(r, g, h):
    m = jnp.mean(r, axis=0)
    v = jnp.mean((r - m[None, :]) ** 2, axis=0)
    return (r - m[None, :]) / jnp.sqrt(v + EPS)[None, :] * g[None, :] \
        + h[None, :]


def _tc0_body(x_ref, g0_ref, b0_ref, W1_ref, y1_ref):
    xn = _bn(x_ref[...], g0_ref[...], b0_ref[...])
    y1_ref[...] = jnp.dot(xn, W1_ref[...], preferred_element_type=jnp.float32)


def _tc_u1_body(y1_ref, degp_ref, dinv_ref, u1_ref):
    deg = degp_ref[0, :N, 0] + degp_ref[1, :N, 0] + 1.0
    dinv = lax.rsqrt(jnp.maximum(deg, 1.0))
    dinv_ref[...] = dinv
    u1_ref[...] = dinv[:, None] * y1_ref[...]


def _tc_layer_body(Sp_ref, u_ref, dinv_ref, c_ref, g_ref, h_ref, W_ref,
                   un_ref):
    dinv = dinv_ref[...]
    S = Sp_ref[0, :N] + Sp_ref[1, :N]
    z = dinv[:, None] * (S + u_ref[...]) + c_ref[...][None, :]
    r = jnp.maximum(z, 0.0)
    xn = _bn(r, g_ref[...], h_ref[...])
    y = jnp.dot(xn, W_ref[...], preferred_element_type=jnp.float32)
    un_ref[...] = dinv[:, None] * y


def _tc_final_body(Sp_ref, u_ref, dinv_ref, batch_ref, c4_ref, g4_ref, h4_ref,
                   W5_ref, c5_ref, g5_ref, h5_ref, W6_ref, c6_ref, out_ref):
    dinv = dinv_ref[...]
    S = Sp_ref[0, :N] + Sp_ref[1, :N]
    z = dinv[:, None] * (S + u_ref[...]) + c4_ref[...][None, :]
    r = jnp.maximum(z, 0.0)
    xn = _bn(r, g4_ref[...], h4_ref[...])

    batch = batch_ref[...]
    gids = lax.broadcasted_iota(jnp.int32, (N, G), 1)
    oh = (batch[:, None] == gids).astype(jnp.float32)
    segsum = lax.dot_general(oh, xn, (((0,), (0,)), ((), ())),
                             preferred_element_type=jnp.float32,
                             precision=lax.Precision.HIGHEST)
    cnt = jnp.sum(oh, axis=0)
    pooled = segsum / jnp.maximum(cnt, 1.0)[:, None]

    t = jnp.maximum(
        jnp.dot(pooled, W5_ref[...], preferred_element_type=jnp.float32)
        + c5_ref[...][None, :], 0.0)
    t = _bn(t, g5_ref[...], h5_ref[...])
    out_ref[...] = jnp.dot(t, W6_ref[...], preferred_element_type=jnp.float32) \
        + c6_ref[...][None, :]


def _f32(shape):
    return jax.ShapeDtypeStruct(shape, jnp.float32)


@functools.cache
def _sc_kernels():
    mesh = plsc.VectorSubcoreMesh(core_axis_name="c", subcore_axis_name="s")
    sc_degree = pl.kernel(
        _sc_degree_body,
        out_type=_f32((NC, NPAD, H)),
        mesh=mesh,
        scratch_types=[
            pltpu.VMEM((CHUNKS_PER_TILE, CHUNK), jnp.int32),
            pltpu.VMEM((CHUNK, H), jnp.float32),
            pltpu.VMEM_SHARED((NPAD, H), jnp.float32),
        ],
    )
    sc_scatter = pl.kernel(
        _sc_scatter_body,
        out_type=_f32((NC, NPAD, H)),
        mesh=mesh,
        scratch_types=[
            pltpu.VMEM((CHUNKS_CORE1 // 4, CHUNK), jnp.int32),
            pltpu.VMEM((CHUNKS_CORE1 // 4, CHUNK), jnp.int32),
            pltpu.VMEM((CHUNK, H), jnp.float32),
            pltpu.VMEM((CHUNK, H), jnp.float32),
            pltpu.VMEM_SHARED((NPAD, H), jnp.float32),
            pltpu.SemaphoreType.DMA,
            pltpu.SemaphoreType.DMA,
        ],
    )
    return sc_degree, sc_scatter


_tc0 = pl.pallas_call(_tc0_body, out_shape=_f32((N, H)))
_tc_u1 = pl.pallas_call(_tc_u1_body, out_shape=(_f32((N,)), _f32((N, H))))
_tc_layer = pl.pallas_call(_tc_layer_body, out_shape=_f32((N, H)))
_tc_final = pl.pallas_call(_tc_final_body, out_shape=_f32((G, OUT)))


def kernel(x, edge_index, batch, g0, b0, W1, c1, g1, h1, W2, c2, g2, h2,
           W3, c3, g3, h3, W4, c4, g4, h4, W5, c5, g5, h5, W6, c6):
    src = edge_index[0]
    dst = edge_index[1]
    pad = E_PAD - E
    src_p = jnp.concatenate([src, jnp.zeros((pad,), jnp.int32)])
    dst_p = jnp.concatenate([dst, jnp.full((pad,), TRASH, jnp.int32)])
    srcm = src_p.reshape(E_PAD // CHUNK, CHUNK)
    dstm = dst_p.reshape(E_PAD // CHUNK, CHUNK)

    sc_degree, sc_scatter = _sc_kernels()
    degp = sc_degree(dstm)
    y1 = _tc0(x, g0, b0, W1)
    dinv, u = _tc_u1(y1, degp)

    for c, g, h, W in ((c1, g1, h1, W2), (c2, g2, h2, W3), (c3, g3, h3, W4)):
        Sp = sc_scatter(u, srcm, dstm)
        u = _tc_layer(Sp, u, dinv, c, g, h, W)

    Sp = sc_scatter(u, srcm, dstm)
    return _tc_final(Sp, u, dinv, batch, c4, g4, h4, W5, c5, g5, h5, W6, c6)


# speedup vs baseline: 1.1101x; 1.1101x over previous
import functools

import jax
import jax.numpy as jnp
from jax import lax
from jax.experimental import pallas as pl
from jax.experimental.pallas import tpu as pltpu
from jax.experimental.pallas import tpu_sc as plsc

N = 10000
E = 320000
H = 128
OUT = 40
G = 64
EPS = 1e-5

NC = 2
NS = 16
NW = NC * NS

CHUNK = 128
CHUNKS_PER_TILE = 80
CHUNKS_CORE0 = 112
CHUNKS_CORE1 = 48
EDGES_PER_TILE = CHUNK * CHUNKS_PER_TILE
E_PAD = CHUNK * NS * (CHUNKS_CORE0 + CHUNKS_CORE1)
NPAD = 10112
ROWS_PER_TILE = NPAD // NS
TRASH = N


def _sc_degree_body(dstm_hbm, out_hbm, idx_v, val_v, hist_sh):
    cid = lax.axis_index("c")
    sid = lax.axis_index("s")
    wid = cid * NS + sid

    @pl.loop(0, CHUNK)
    def _zero(i):
        for j in range(H // 16):
            val_v[i, pl.ds(j * 16, 16)] = jnp.zeros((16,), jnp.float32)

    r0 = sid * ROWS_PER_TILE
    for k in range(4):
        pltpu.sync_copy(val_v, hist_sh.at[pl.ds(r0 + k * 128, 128)])
    pltpu.sync_copy(val_v.at[pl.ds(0, ROWS_PER_TILE - 512)],
                    hist_sh.at[pl.ds(r0 + 512, ROWS_PER_TILE - 512)])
    plsc.subcore_barrier()

    @pl.loop(0, CHUNK)
    def _ones(i):
        val_v[i, pl.ds(0, 16)] = jnp.full((16,), 1.0, jnp.float32)

    pltpu.sync_copy(dstm_hbm.at[pl.ds(wid * CHUNKS_PER_TILE, CHUNKS_PER_TILE)],
                    idx_v)

    @pl.loop(0, CHUNKS_PER_TILE)
    def _accum(ch):
        pltpu.sync_copy(val_v, hist_sh.at[idx_v.at[ch]], add=True)

    plsc.subcore_barrier()
    pltpu.sync_copy(hist_sh.at[pl.ds(r0, ROWS_PER_TILE)],
                    out_hbm.at[cid, pl.ds(r0, ROWS_PER_TILE)])


def _sc_scatter_body(u_hbm, srcm_hbm, dstm_hbm, out_hbm,
                     src_v, dst_v, row_a, row_b, acc_sh, sem_a, sem_b):
    cid = lax.axis_index("c")
    sid = lax.axis_index("s")
    wid = cid * NS + sid

    @pl.loop(0, CHUNK)
    def _zero(i):
        for j in range(H // 16):
            row_a[i, pl.ds(j * 16, 16)] = jnp.zeros((16,), jnp.float32)

    r0 = sid * ROWS_PER_TILE
    for k in range(4):
        pltpu.sync_copy(row_a, acc_sh.at[pl.ds(r0 + k * 128, 128)])
    pltpu.sync_copy(row_a.at[pl.ds(0, ROWS_PER_TILE - 512)],
                    acc_sh.at[pl.ds(r0 + 512, ROWS_PER_TILE - 512)])
    plsc.subcore_barrier()

    def _gather_start(ch, buf, sem):
        pltpu.async_copy(u_hbm.at[src_v.at[ch]], buf, sem)

    def _gather_wait(ch, buf, sem):
        pltpu.make_async_copy(u_hbm.at[src_v.at[ch]], buf, sem).wait()

    def _run_edges(nchunks, e0):
        q = nchunks // 2
        for hb in range(0, nchunks, q):
            pltpu.sync_copy(srcm_hbm.at[pl.ds(e0 + hb, q)],
                            src_v.at[pl.ds(0, q)])
            pltpu.sync_copy(dstm_hbm.at[pl.ds(e0 + hb, q)],
                            dst_v.at[pl.ds(0, q)])
            _gather_start(0, row_a, sem_a)

            @pl.loop(0, q // 2)
            def _edges(p):
                ch = p * 2
                _gather_start(ch + 1, row_b, sem_b)
                _gather_wait(ch, row_a, sem_a)
                pltpu.sync_copy(row_a, acc_sh.at[dst_v.at[ch]], add=True)

                @pl.when(ch + 2 < q)
                def _():
                    _gather_start(ch + 2, row_a, sem_a)

                _gather_wait(ch + 1, row_b, sem_b)
                pltpu.sync_copy(row_b, acc_sh.at[dst_v.at[ch + 1]], add=True)

    @pl.when(cid == 0)
    def _core0():
        _run_edges(CHUNKS_CORE0, sid * CHUNKS_CORE0)

    @pl.when(cid == 1)
    def _core1():
        _run_edges(CHUNKS_CORE1, NS * CHUNKS_CORE0 + sid * CHUNKS_CORE1)

    plsc.subcore_barrier()
    pltpu.sync_copy(acc_sh.at[pl.ds(r0, ROWS_PER_TILE)],
                    out_hbm.at[cid, pl.ds(r0, ROWS_PER_TILE)])


def _bn(r, g, h):
    m = jnp.mean(r, axis=0)
    v = jnp.mean((r - m[None, :]) ** 2, axis=0)
    return (r - m[None, :]) / jnp.sqrt(v + EPS)[None, :] * g[None, :] \
        + h[None, :]


def _tc0_body(x_ref, g0_ref, b0_ref, W1_ref, y1_ref):
    xn = _bn(x_ref[...], g0_ref[...], b0_ref[...])
    y1_ref[...] = jnp.dot(xn, W1_ref[...], preferred_element_type=jnp.float32)


def _tc_u1_body(y1_ref, degp_ref, dinv_ref, u1_ref):
    deg = degp_ref[0, :N, 0] + degp_ref[1, :N, 0] + 1.0
    dinv = lax.rsqrt(jnp.maximum(deg, 1.0))
    dinv_ref[...] = dinv
    u1_ref[...] = dinv[:, None] * y1_ref[...]


def _tc_layer_body(Sp_ref, u_ref, dinv_ref, c_ref, g_ref, h_ref, W_ref,
                   un_ref):
    dinv = dinv_ref[...]
    S = Sp_ref[0, :N] + Sp_ref[1, :N]
    z = dinv[:, None] * (S + u_ref[...]) + c_ref[...][None, :]
    r = jnp.maximum(z, 0.0)
    xn = _bn(r, g_ref[...], h_ref[...])
    y = jnp.dot(xn, W_ref[...], preferred_element_type=jnp.float32)
    un_ref[...] = dinv[:, None] * y


def _tc_final_body(Sp_ref, u_ref, dinv_ref, batch_ref, c4_ref, g4_ref, h4_ref,
                   W5_ref, c5_ref, g5_ref, h5_ref, W6_ref, c6_ref, out_ref):
    dinv = dinv_ref[...]
    S = Sp_ref[0, :N] + Sp_ref[1, :N]
    z = dinv[:, None] * (S + u_ref[...]) + c4_ref[...][None, :]
    r = jnp.maximum(z, 0.0)
    xn = _bn(r, g4_ref[...], h4_ref[...])

    batch = batch_ref[...]
    gids = lax.broadcasted_iota(jnp.int32, (N, G), 1)
    oh = (batch[:, None] == gids).astype(jnp.float32)
    segsum = lax.dot_general(oh, xn, (((0,), (0,)), ((), ())),
                             preferred_element_type=jnp.float32,
                             precision=lax.Precision.HIGHEST)
    cnt = jnp.sum(oh, axis=0)
    pooled = segsum / jnp.maximum(cnt, 1.0)[:, None]

    t = jnp.maximum(
        jnp.dot(pooled, W5_ref[...], preferred_element_type=jnp.float32)
        + c5_ref[...][None, :], 0.0)
    t = _bn(t, g5_ref[...], h5_ref[...])
    out_ref[...] = jnp.dot(t, W6_ref[...], preferred_element_type=jnp.float32) \
        + c6_ref[...][None, :]


def _f32(shape):
    return jax.ShapeDtypeStruct(shape, jnp.float32)


@functools.cache
def _sc_kernels():
    mesh = plsc.VectorSubcoreMesh(core_axis_name="c", subcore_axis_name="s")
    sc_degree = pl.kernel(
        _sc_degree_body,
        out_type=_f32((NC, NPAD, H)),
        mesh=mesh,
        scratch_types=[
            pltpu.VMEM((CHUNKS_PER_TILE, CHUNK), jnp.int32),
            pltpu.VMEM((CHUNK, H), jnp.float32),
            pltpu.VMEM_SHARED((NPAD, H), jnp.float32),
        ],
    )
    sc_scatter = pl.kernel(
        _sc_scatter_body,
        out_type=_f32((NC, NPAD, H)),
        mesh=mesh,
        scratch_types=[
            pltpu.VMEM((CHUNKS_CORE0 // 2, CHUNK), jnp.int32),
            pltpu.VMEM((CHUNKS_CORE0 // 2, CHUNK), jnp.int32),
            pltpu.VMEM((CHUNK, H), jnp.float32),
            pltpu.VMEM((CHUNK, H), jnp.float32),
            pltpu.VMEM_SHARED((NPAD, H), jnp.float32),
            pltpu.SemaphoreType.DMA,
            pltpu.SemaphoreType.DMA,
        ],
    )
    return sc_degree, sc_scatter


_tc0 = pl.pallas_call(_tc0_body, out_shape=_f32((N, H)))
_tc_u1 = pl.pallas_call(_tc_u1_body, out_shape=(_f32((N,)), _f32((N, H))))
_tc_layer = pl.pallas_call(_tc_layer_body, out_shape=_f32((N, H)))
_tc_final = pl.pallas_call(_tc_final_body, out_shape=_f32((G, OUT)))


def kernel(x, edge_index, batch, g0, b0, W1, c1, g1, h1, W2, c2, g2, h2,
           W3, c3, g3, h3, W4, c4, g4, h4, W5, c5, g5, h5, W6, c6):
    src = edge_index[0]
    dst = edge_index[1]
    pad = E_PAD - E
    src_p = jnp.concatenate([src, jnp.zeros((pad,), jnp.int32)])
    dst_p = jnp.concatenate([dst, jnp.full((pad,), TRASH, jnp.int32)])
    srcm = src_p.reshape(E_PAD // CHUNK, CHUNK)
    dstm = dst_p.reshape(E_PAD // CHUNK, CHUNK)

    sc_degree, sc_scatter = _sc_kernels()
    degp = sc_degree(dstm)
    y1 = _tc0(x, g0, b0, W1)
    dinv, u = _tc_u1(y1, degp)

    for c, g, h, W in ((c1, g1, h1, W2), (c2, g2, h2, W3), (c3, g3, h3, W4)):
        Sp = sc_scatter(u, srcm, dstm)
        u = _tc_layer(Sp, u, dinv, c, g, h, W)

    Sp = sc_scatter(u, srcm, dstm)
    return _tc_final(Sp, u, dinv, batch, c4, g4, h4, W5, c5, g5, h5, W6, c6)
